# baseline (speedup 1.0000x reference)
import jax
import jax.numpy as jnp
from jax import lax
from jax.experimental import pallas as pl


def _body(ts_ref, x_ref, peB_ref, peBs_ref, peA_ref, peAs_ref, emb_ref,
          out_ref):
    S = x_ref.shape[1]
    D = x_ref.shape[2]
    C = D // 4
    ts = ts_ref[0]
    F = peB_ref[...]
    Fs = peBs_ref[...]
    even = (lax.broadcasted_iota(jnp.int32, (256, D), 1) % 2) == 0
    for k in range(S // 256):
        E = peA_ref[k, :][None, :]
        Es = peAs_ref[k, :][None, :]
        pe_k = (jnp.where(even, Es * F, E * F)
                + jnp.where(even, E * Fs, -(Es * Fs)))
        r0 = k * 256
        xb = x_ref[0, r0:r0 + 256, :]
        for c in range(4):
            idx = ts[c, r0:r0 + 256]
            oh = (idx[:, None] ==
                  lax.broadcasted_iota(jnp.int32, (256, 32), 1))
            chunk = jnp.dot(oh.astype(jnp.float32),
                            emb_ref[:, c * C:(c + 1) * C],
                            preferred_element_type=jnp.float32)
            out_ref[0, r0:r0 + 256, c * C:(c + 1) * C] = (
                xb[:, c * C:(c + 1) * C] + pe_k[:, c * C:(c + 1) * C] + chunk)


def kernel(x, timestamps, pe, hour_emb, day_emb, month_emb, season_emb):
    B, L, D = x.shape
    S = 2048
    nsb = L // S

    pe2 = pe[0]
    tsT = timestamps.transpose(0, 2, 1)

    def swap_pairs(a):
        a3 = a.reshape(a.shape[0], D // 2, 2)
        return jnp.concatenate([a3[:, :, 1:2], a3[:, :, 0:1]],
                               axis=2).reshape(a.shape[0], D)

    peBs = swap_pairs(pe2[:256])
    peA = pe2[0:S:256]
    peAs = swap_pairs(peA)

    def pad32(e):
        return jnp.pad(e, ((0, 32 - e.shape[0]), (0, 0)))

    emb = jnp.concatenate(
        [pad32(hour_emb), pad32(day_emb), pad32(month_emb), pad32(season_emb)],
        axis=1)

    KA = S // 256
    return pl.pallas_call(
        _body,
        grid=(nsb, B),
        in_specs=[
            pl.BlockSpec((1, 4, S), lambda i, j: (j, 0, i)),
            pl.BlockSpec((1, S, D), lambda i, j: (j, i, 0)),
            pl.BlockSpec((256, D), lambda i, j: (0, 0)),
            pl.BlockSpec((256, D), lambda i, j: (0, 0)),
            pl.BlockSpec((KA, D), lambda i, j: (0, 0)),
            pl.BlockSpec((KA, D), lambda i, j: (0, 0)),
            pl.BlockSpec((32, D), lambda i, j: (0, 0)),
        ],
        out_specs=pl.BlockSpec((1, S, D), lambda i, j: (j, i, 0)),
        out_shape=jax.ShapeDtypeStruct((B, L, D), x.dtype),
    )(tsT, x, pe2, peBs, peA, peAs, emb)


# speedup vs baseline: 1.5115x; 1.5115x over previous
import jax
import jax.numpy as jnp
from jax import lax
from jax.experimental import pallas as pl


def _body(ts_ref, x_ref, pe_ref, emb_ref, out_ref):
    S = x_ref.shape[1]
    D = x_ref.shape[2]
    C = D // 4
    ts = ts_ref[0]
    xb = x_ref[0]
    peb = pe_ref[...]
    for c in range(4):
        idx = ts[c, :]
        oh = (idx[:, None] == lax.broadcasted_iota(jnp.int32, (S, 32), 1))
        chunk = jnp.dot(oh.astype(jnp.float32),
                        emb_ref[:, c * C:(c + 1) * C],
                        preferred_element_type=jnp.float32)
        out_ref[0, :, c * C:(c + 1) * C] = (
            xb[:, c * C:(c + 1) * C] + peb[:, c * C:(c + 1) * C] + chunk)


def kernel(x, timestamps, pe, hour_emb, day_emb, month_emb, season_emb):
    B, L, D = x.shape
    C = D // 4
    S = 2048
    nsb = L // S

    pe2 = pe[0]
    tsT = timestamps.transpose(0, 2, 1)

    def pad32(e):
        return jnp.pad(e, ((0, 32 - e.shape[0]), (0, 0)))

    emb = jnp.concatenate(
        [pad32(hour_emb), pad32(day_emb), pad32(month_emb), pad32(season_emb)],
        axis=1)

    return pl.pallas_call(
        _body,
        grid=(nsb, B),
        in_specs=[
            pl.BlockSpec((1, 4, S), lambda i, j: (j, 0, i)),
            pl.BlockSpec((1, S, D), lambda i, j: (j, i, 0)),
            pl.BlockSpec((S, D), lambda i, j: (i, 0)),
            pl.BlockSpec((32, D), lambda i, j: (0, 0)),
        ],
        out_specs=pl.BlockSpec((1, S, D), lambda i, j: (j, i, 0)),
        out_shape=jax.ShapeDtypeStruct((B, L, D), x.dtype),
    )(tsT, x, pe2, emb)
